# baseline (speedup 1.0000x reference)
import jax
import jax.numpy as jnp
from jax import lax
from jax.experimental import pallas as pl
from jax.experimental.pallas import tpu as pltpu
from jax.experimental.pallas import tpu_sc as plsc

_VOCAB = 50257
_DIM = 768
_BATCH = 64
_SEQ = 1024
_TOK = _BATCH * _SEQ

_NC = 2
_NS = 16
_NW = _NC * _NS
_TPW = _TOK // _NW
_CH = 32
_NCH = _TPW // _CH
_NB = 4
_LANES = 16
_DREGS = _DIM // _LANES
_GRP = _TPW // _LANES


def _body(ids_hbm, mask_hbm, table_hbm, out_hbm,
          idx_v, mask_v, rows_v, gsem0, gsem1, gsem2, gsem3,
          osem0, osem1, osem2, osem3):
    wid = lax.axis_index("c") * _NS + lax.axis_index("s")
    base = wid * _TPW
    gsems = (gsem0, gsem1, gsem2, gsem3)
    osems = (osem0, osem1, osem2, osem3)

    pltpu.sync_copy(ids_hbm.at[wid], idx_v)
    pltpu.sync_copy(mask_hbm.at[wid], mask_v)

    def fold(k, _):
        for h in range(_CH // _LANES):
            sl = pl.ds(h * _LANES, _LANES)
            idx_v[k, sl] = idx_v[k, sl] * mask_v[k, sl]
        return 0

    lax.fori_loop(0, _NCH, fold, 0, unroll=False)

    def gather_desc(i, b):
        return pltpu.make_async_copy(
            table_hbm.at[idx_v.at[i]], rows_v.at[b], gsems[b])

    def out_desc(i, b):
        return pltpu.make_async_copy(
            rows_v.at[b], out_hbm.at[pl.ds(base + i * _CH, _CH)], osems[b])

    gather_desc(0, 0).start()
    gather_desc(1, 1).start()

    def chunk(i, b, b2):
        gather_desc(i, b).wait()

        def group(g, _):
            gbase = g * _LANES
            m16 = mask_v[i, pl.ds(gbase, _LANES)]
            zero = jnp.zeros((_LANES,), jnp.float32)
            for t in range(_LANES):
                @pl.when(m16[t] == 0)
                def _():
                    for j in range(_DREGS):
                        rows_v[b, gbase + t, pl.ds(j * _LANES, _LANES)] = zero
            return 0

        lax.fori_loop(0, _CH // _LANES, group, 0, unroll=False)

        out_desc(i, b).start()

        @pl.when(i >= 2)
        def _():
            out_desc(i - 2, b2).wait()

        @pl.when(i + 2 < _NCH)
        def _():
            gather_desc(i + 2, b2).start()

    def quad(k, _):
        io = k * _NB
        chunk(io + 0, 0, 2)
        chunk(io + 1, 1, 3)
        chunk(io + 2, 2, 0)
        chunk(io + 3, 3, 1)
        return 0

    lax.fori_loop(0, _NCH // _NB, quad, 0, unroll=False)

    out_desc(_NCH - 2, (_NCH - 2) % _NB).wait()
    out_desc(_NCH - 1, (_NCH - 1) % _NB).wait()


@jax.jit
def _lookup(ids, mask_i, table):
    mesh = plsc.VectorSubcoreMesh(core_axis_name="c", subcore_axis_name="s")
    run = pl.kernel(
        _body,
        out_type=jax.ShapeDtypeStruct((_TOK, _DIM), jnp.float32),
        mesh=mesh,
        scratch_types=[
            pltpu.VMEM((_NCH, _CH), jnp.int32),
            pltpu.VMEM((_NCH, _CH), jnp.int32),
            pltpu.VMEM((_NB, _CH, _DIM), jnp.float32),
            pltpu.SemaphoreType.DMA,
            pltpu.SemaphoreType.DMA,
            pltpu.SemaphoreType.DMA,
            pltpu.SemaphoreType.DMA,
            pltpu.SemaphoreType.DMA,
            pltpu.SemaphoreType.DMA,
            pltpu.SemaphoreType.DMA,
            pltpu.SemaphoreType.DMA,
        ],
    )
    return run(ids, mask_i, table)


def kernel(input_ids, attention_mask, table):
    ids = input_ids.reshape(_NW, _NCH, _CH).astype(jnp.int32)
    mask_i = attention_mask.reshape(_NW, _NCH, _CH).astype(jnp.int32)
    out = _lookup(ids, mask_i, table)
    return out.reshape(_BATCH, _SEQ, _DIM), attention_mask


# speedup vs baseline: 9.1359x; 9.1359x over previous
import jax
import jax.numpy as jnp
from jax import lax
from jax.experimental import pallas as pl
from jax.experimental.pallas import tpu as pltpu
from jax.experimental.pallas import tpu_sc as plsc

_VOCAB = 50257
_DIM = 768
_BATCH = 64
_SEQ = 1024
_TOK = _BATCH * _SEQ

_NC = 2
_NS = 16
_NW = _NC * _NS
_TPW = _TOK // _NW
_CH = 32
_NCH = _TPW // _CH
_NB = 4
_LANES = 16
_DREGS = _DIM // _LANES
_GRP = _TPW // _LANES


def _body(ids_hbm, mask_hbm, table_hbm, out_hbm,
          idx_v, mask_v, rows_v, gsem0, gsem1, gsem2, gsem3,
          osem0, osem1, osem2, osem3):
    wid = lax.axis_index("c") * _NS + lax.axis_index("s")
    base = wid * _TPW
    gsems = (gsem0, gsem1, gsem2, gsem3)
    osems = (osem0, osem1, osem2, osem3)

    pltpu.sync_copy(ids_hbm.at[wid], idx_v)
    pltpu.sync_copy(mask_hbm.at[wid], mask_v)

    def fold(k, _):
        for h in range(_CH // _LANES):
            sl = pl.ds(h * _LANES, _LANES)
            idx_v[k, sl] = idx_v[k, sl] * mask_v[k, sl]
        return 0


    def gather_desc(i, b):
        return pltpu.make_async_copy(
            table_hbm.at[idx_v.at[i]], rows_v.at[b], gsems[b])

    def out_desc(i, b):
        return pltpu.make_async_copy(
            rows_v.at[b], out_hbm.at[pl.ds(base + i * _CH, _CH)], osems[b])

    gather_desc(0, 0).start()
    gather_desc(1, 1).start()

    def chunk(i, b, b2):
        gather_desc(i, b).wait()

        def group(g, _):
            gbase = g * _LANES
            m16 = mask_v[i, pl.ds(gbase, _LANES)]
            zero = jnp.zeros((_LANES,), jnp.float32)
            for t in range(_LANES):
                @pl.when(m16[t] == 0)
                def _():
                    for j in range(_DREGS):
                        rows_v[b, gbase + t, pl.ds(j * _LANES, _LANES)] = zero
            return 0

        lax.fori_loop(0, _CH // _LANES, group, 0, unroll=False)

        out_desc(i, b).start()

        @pl.when(i >= 2)
        def _():
            out_desc(i - 2, b2).wait()

        @pl.when(i + 2 < _NCH)
        def _():
            gather_desc(i + 2, b2).start()

    def quad(k, _):
        io = k * _NB
        chunk(io + 0, 0, 2)
        chunk(io + 1, 1, 3)
        chunk(io + 2, 2, 0)
        chunk(io + 3, 3, 1)
        return 0

    lax.fori_loop(0, _NCH // _NB, quad, 0, unroll=False)

    out_desc(_NCH - 2, (_NCH - 2) % _NB).wait()
    out_desc(_NCH - 1, (_NCH - 1) % _NB).wait()


@jax.jit
def _lookup(ids, mask_i, table):
    mesh = plsc.VectorSubcoreMesh(core_axis_name="c", subcore_axis_name="s")
    run = pl.kernel(
        _body,
        out_type=jax.ShapeDtypeStruct((_TOK, _DIM), jnp.float32),
        mesh=mesh,
        scratch_types=[
            pltpu.VMEM((_NCH, _CH), jnp.int32),
            pltpu.VMEM((_NCH, _CH), jnp.int32),
            pltpu.VMEM((_NB, _CH, _DIM), jnp.float32),
            pltpu.SemaphoreType.DMA,
            pltpu.SemaphoreType.DMA,
            pltpu.SemaphoreType.DMA,
            pltpu.SemaphoreType.DMA,
            pltpu.SemaphoreType.DMA,
            pltpu.SemaphoreType.DMA,
            pltpu.SemaphoreType.DMA,
            pltpu.SemaphoreType.DMA,
        ],
    )
    return run(ids, mask_i, table)


def kernel(input_ids, attention_mask, table):
    ids = input_ids.reshape(_NW, _NCH, _CH).astype(jnp.int32)
    mask_i = attention_mask.reshape(_NW, _NCH, _CH).astype(jnp.int32)
    out = _lookup(ids, mask_i, table)
    return out.reshape(_BATCH, _SEQ, _DIM), attention_mask


# speedup vs baseline: 11.5836x; 1.2679x over previous
import jax
import jax.numpy as jnp
from jax import lax
from jax.experimental import pallas as pl
from jax.experimental.pallas import tpu as pltpu
from jax.experimental.pallas import tpu_sc as plsc

_VOCAB = 50257
_DIM = 768
_BATCH = 64
_SEQ = 1024
_TOK = _BATCH * _SEQ

_NC = 2
_NS = 16
_NW = _NC * _NS
_TPW = _TOK // _NW
_LANES = 16
_CH = _LANES
_GRP = _TPW // _LANES
_ROWS = _GRP + 2
_BUF = _ROWS * _LANES
_DREGS = _DIM // _LANES
_ZWIN = 8


def _body(ids_hbm, mask_hbm, table_hbm, out_hbm,
          ids_v, uslot_v, ids_c, uslot_c, mslot_c, gid2, uslot2, mslot2,
          rbuf, zbuf,
          gsem0, gsem1, gsem2, gsem3, osem0, osem1, osem2, osem3, zsem):
    wid = lax.axis_index("c") * _NS + lax.axis_index("s")
    base = wid * _TPW
    gsems = (gsem0, gsem1, gsem2, gsem3)
    osems = (osem0, osem1, osem2, osem3)
    zero16 = jnp.zeros((_LANES,), jnp.float32)

    pltpu.sync_copy(ids_hbm.at[wid], ids_v)
    pltpu.sync_copy(mask_hbm.at[wid], uslot_v)

    def zrow(r, _):
        for j in range(_DREGS):
            zbuf[r, pl.ds(j * _LANES, _LANES)] = zero16
        return 0

    lax.fori_loop(0, _CH, zrow, 0, unroll=False)

    def part(g, carry):
        n1, n0 = carry
        sl = pl.ds(g * _LANES, _LANES)
        id16 = ids_v[sl]
        m16 = uslot_v[sl]
        slot0 = base + g * _LANES
        for t in range(_LANES):
            mt = m16[t]
            ids_c[pl.ds(n1, _LANES)] = jnp.full((_LANES,), id16[t], jnp.int32)
            uslot_c[pl.ds(n1, _LANES)] = jnp.full((_LANES,), slot0 + t,
                                                  jnp.int32)
            mslot_c[pl.ds(n0, _LANES)] = jnp.full((_LANES,), slot0 + t,
                                                  jnp.int32)
            n1 = n1 + mt
            n0 = n0 + (1 - mt)
        return n1, n0

    n1, n0 = lax.fori_loop(0, _GRP, part, (jnp.int32(0), jnp.int32(0)),
                           unroll=False)

    @pl.when(n1 > 0)
    def _():
        last_id = ids_c[pl.ds(n1 - 1, _LANES)][0]
        last_sl = uslot_c[pl.ds(n1 - 1, _LANES)][0]
        ids_c[pl.ds(n1, _LANES)] = jnp.full((_LANES,), last_id, jnp.int32)
        uslot_c[pl.ds(n1, _LANES)] = jnp.full((_LANES,), last_sl, jnp.int32)

    @pl.when(n0 > 0)
    def _():
        last_ms = mslot_c[pl.ds(n0 - 1, _LANES)][0]
        mslot_c[pl.ds(n0, _LANES)] = jnp.full((_LANES,), last_ms, jnp.int32)

    def relay(r, _):
        sl = pl.ds(r * _LANES, _LANES)
        gid2[r, :] = ids_c[sl]
        uslot2[r, :] = uslot_c[sl]
        mslot2[r, :] = mslot_c[sl]
        return 0

    lax.fori_loop(0, _ROWS, relay, 0, unroll=False)

    c1 = (n1 + _CH - 1) // _CH
    c0 = (n0 + _CH - 1) // _CH

    def gdesc(k, b):
        return pltpu.make_async_copy(table_hbm.at[gid2.at[k]], rbuf.at[b],
                                     gsems[b])

    def sdesc(k, b):
        return pltpu.make_async_copy(rbuf.at[b], out_hbm.at[uslot2.at[k]],
                                     osems[b])

    def zdesc(k):
        return pltpu.make_async_copy(zbuf, out_hbm.at[mslot2.at[k]], zsem)

    for b in range(2):
        @pl.when(b < c1)
        def _(b=b):
            gdesc(b, b).start()

    def quad(q, _):
        for b in range(4):
            k = q * 4 + b
            b2 = (b + 2) % 4

            @pl.when(k < c1)
            def _(k=k, b=b, b2=b2):
                gdesc(k, b).wait()
                sdesc(k, b).start()

                @pl.when(k >= 2)
                def _():
                    sdesc(k - 2, b2).wait()

                @pl.when(k + 2 < c1)
                def _():
                    gdesc(k + 2, b2).start()

            @pl.when(k < c0)
            def _(k=k):
                zdesc(k).start()

                @pl.when(k >= _ZWIN)
                def _():
                    zdesc(k - _ZWIN).wait()
        return 0

    nq = (jnp.maximum(c1, c0) + 3) // 4
    lax.fori_loop(0, nq, quad, 0, unroll=False)

    for b in range(4):
        tail = ((c1 >= 1) & ((c1 - 1) % 4 == b)) | \
               ((c1 >= 2) & ((c1 - 2) % 4 == b))

        @pl.when(tail)
        def _(b=b):
            sdesc(0, b).wait()

    lax.fori_loop(0, jnp.minimum(c0, _ZWIN), lambda i, _: (zdesc(0).wait(), 0)[1],
                  0, unroll=False)


@jax.jit
def _lookup(ids, mask_i, table):
    mesh = plsc.VectorSubcoreMesh(core_axis_name="c", subcore_axis_name="s")
    run = pl.kernel(
        _body,
        out_type=jax.ShapeDtypeStruct((_TOK, _DIM), jnp.float32),
        mesh=mesh,
        scratch_types=[
            pltpu.VMEM((_TPW,), jnp.int32),
            pltpu.VMEM((_TPW,), jnp.int32),
            pltpu.VMEM((_BUF,), jnp.int32),
            pltpu.VMEM((_BUF,), jnp.int32),
            pltpu.VMEM((_BUF,), jnp.int32),
            pltpu.VMEM((_ROWS, _CH), jnp.int32),
            pltpu.VMEM((_ROWS, _CH), jnp.int32),
            pltpu.VMEM((_ROWS, _CH), jnp.int32),
            pltpu.VMEM((4, _CH, _DIM), jnp.float32),
            pltpu.VMEM((_CH, _DIM), jnp.float32),
            pltpu.SemaphoreType.DMA,
            pltpu.SemaphoreType.DMA,
            pltpu.SemaphoreType.DMA,
            pltpu.SemaphoreType.DMA,
            pltpu.SemaphoreType.DMA,
            pltpu.SemaphoreType.DMA,
            pltpu.SemaphoreType.DMA,
            pltpu.SemaphoreType.DMA,
            pltpu.SemaphoreType.DMA,
        ],
    )
    return run(ids, mask_i, table)


def kernel(input_ids, attention_mask, table):
    ids = input_ids.reshape(_NW, _TPW).astype(jnp.int32)
    mask_i = attention_mask.reshape(_NW, _TPW).astype(jnp.int32)
    out = _lookup(ids, mask_i, table)
    return out.reshape(_BATCH, _SEQ, _DIM), attention_mask
